# baseline (speedup 1.0000x reference)
import functools

import jax
import jax.numpy as jnp
from jax import lax
from jax.experimental import pallas as pl
from jax.experimental.pallas import tpu as pltpu
from jax.experimental.pallas import tpu_sc as plsc

N = 10000
E = 320000
D = 128
H = 128
C = 16
G = 64
EPS = 1e-5

NC = 2
NS = 16
NW = NC * NS
EPW = E // NW
K = 100
NCH = EPW // K
HH = H // 2
EPT = E // NS
ACH = EPT // K
STR = 624
LAST = N - (NS - 1) * STR
DW = 16

_mesh = plsc.VectorSubcoreMesh(core_axis_name="c", subcore_axis_name="s")


@functools.partial(
    pl.kernel,
    out_type=jax.ShapeDtypeStruct((NC, N, DW), jnp.float32),
    mesh=_mesh,
    scratch_types=[
        pltpu.VMEM((NCH, K), jnp.int32),
        pltpu.VMEM((K, DW), jnp.float32),
        pltpu.VMEM_SHARED((N, DW), jnp.float32),
        pltpu.SemaphoreType.DMA,
        pltpu.SemaphoreType.DMA,
    ],
    compiler_params=pltpu.CompilerParams(use_tc_tiling_on_sc=False),
)
def _deg_kernel(dst_hbm, ones_hbm, zeros_hbm, dpart_hbm, dst_v, ones_v, accd,
                s0, s1):
    cid = lax.axis_index("c")
    sid = lax.axis_index("s")
    row0 = sid * STR
    pltpu.sync_copy(dst_hbm.at[sid, pl.ds(cid * NCH, NCH)], dst_v)
    pltpu.sync_copy(ones_hbm, ones_v)

    @pl.when(sid < NS - 1)
    def _():
        pltpu.sync_copy(zeros_hbm.at[pl.ds(0, STR)],
                        accd.at[pl.ds(row0, STR)])

    @pl.when(sid == NS - 1)
    def _():
        pltpu.sync_copy(zeros_hbm, accd.at[pl.ds((NS - 1) * STR, LAST)])

    plsc.subcore_barrier()

    def body(jj, carry):
        c0 = pltpu.async_copy(ones_v, accd.at[dst_v.at[2 * jj]], s0, add=True)
        c1 = pltpu.async_copy(ones_v, accd.at[dst_v.at[2 * jj + 1]], s1,
                              add=True)
        c0.wait()
        c1.wait()
        return carry

    lax.fori_loop(0, NCH // 2, body, 0)
    plsc.subcore_barrier()

    @pl.when(sid < NS - 1)
    def _():
        pltpu.sync_copy(accd.at[pl.ds(row0, STR)],
                        dpart_hbm.at[cid, pl.ds(row0, STR)])

    @pl.when(sid == NS - 1)
    def _():
        pltpu.sync_copy(accd.at[pl.ds((NS - 1) * STR, LAST)],
                        dpart_hbm.at[cid, pl.ds((NS - 1) * STR, LAST)])


@functools.partial(
    pl.kernel,
    out_type=(jax.ShapeDtypeStruct((N, HH), jnp.bfloat16),
              jax.ShapeDtypeStruct((N, HH), jnp.bfloat16)),
    mesh=_mesh,
    scratch_types=[
        pltpu.VMEM((ACH, K), jnp.int32),
        pltpu.VMEM((ACH, K), jnp.int32),
    ] + [pltpu.VMEM((K, HH), jnp.bfloat16)] * 5
      + [pltpu.VMEM_SHARED((N, HH), jnp.bfloat16)]
      + [pltpu.SemaphoreType.DMA] * 10,
    compiler_params=pltpu.CompilerParams(use_tc_tiling_on_sc=False),
)
def _agg_kernel(ga_hbm, gb_hbm, src_hbm, dst_hbm,
                pa_hbm, pb_hbm,
                src_v, dst_v, *rest):
    bufs = rest[0:5]
    acc = rest[5]
    gsems = rest[6:11]
    ssems = rest[11:16]
    cid = lax.axis_index("c")
    sid = lax.axis_index("s")
    row0 = sid * STR
    pltpu.sync_copy(src_hbm.at[sid], src_v)
    pltpu.sync_copy(dst_hbm.at[sid], dst_v)

    @pl.when((cid == 0) & (sid < NS - 1))
    def _():
        pltpu.sync_copy(ga_hbm.at[pl.ds(row0, STR)], acc.at[pl.ds(row0, STR)])

    @pl.when((cid == 0) & (sid == NS - 1))
    def _():
        pltpu.sync_copy(ga_hbm.at[pl.ds((NS - 1) * STR, LAST)],
                        acc.at[pl.ds((NS - 1) * STR, LAST)])

    @pl.when((cid != 0) & (sid < NS - 1))
    def _():
        pltpu.sync_copy(gb_hbm.at[pl.ds(row0, STR)], acc.at[pl.ds(row0, STR)])

    @pl.when((cid != 0) & (sid == NS - 1))
    def _():
        pltpu.sync_copy(gb_hbm.at[pl.ds((NS - 1) * STR, LAST)],
                        acc.at[pl.ds((NS - 1) * STR, LAST)])

    plsc.subcore_barrier()

    NB = 5

    def gather(j, buf, sem):
        @pl.when(cid == 0)
        def _():
            pltpu.async_copy(ga_hbm.at[src_v.at[j]], buf, sem)

        @pl.when(cid != 0)
        def _():
            pltpu.async_copy(gb_hbm.at[src_v.at[j]], buf, sem)

    for b in range(NB):
        gather(b, bufs[b], gsems[b])

    def body(jj, carry):
        scs = []
        for b in range(NB):
            j = NB * jj + b
            pltpu.make_async_copy(ga_hbm.at[src_v.at[0]], bufs[b],
                                  gsems[b]).wait()
            scs.append(pltpu.async_copy(bufs[b], acc.at[dst_v.at[j]],
                                        ssems[b], add=True))
        for b in range(NB):
            j = NB * jj + b
            scs[b].wait()

            @pl.when(j + NB < ACH)
            def _(b=b, j=j):
                gather(j + NB, bufs[b], gsems[b])

        return carry

    lax.fori_loop(0, ACH // NB, body, 0)
    plsc.subcore_barrier()

    def writeback(out_hbm):
        @pl.when(sid < NS - 1)
        def _():
            pltpu.sync_copy(acc.at[pl.ds(row0, STR)],
                            out_hbm.at[pl.ds(row0, STR)])

        @pl.when(sid == NS - 1)
        def _():
            pltpu.sync_copy(acc.at[pl.ds((NS - 1) * STR, LAST)],
                            out_hbm.at[pl.ds((NS - 1) * STR, LAST)])

    @pl.when(cid == 0)
    def _():
        writeback(pa_hbm)

    @pl.when(cid != 0)
    def _():
        writeback(pb_hbm)


BN = 5000
NBLK = N // BN


def _mm_body(x_ref, w_ref, dp_ref, ga_ref, gb_ref):
    deg = dp_ref[0, :, 0:1] + dp_ref[1, :, 0:1] + 1.0
    dinv = lax.rsqrt(deg)
    h = jnp.dot(x_ref[...], w_ref[...], preferred_element_type=jnp.float32,
                precision=lax.Precision.HIGHEST)
    g = (h * dinv).astype(jnp.bfloat16)
    ga_ref[...] = g[:, :HH]
    gb_ref[...] = g[:, HH:]


_matmul_scale = pl.pallas_call(
    _mm_body,
    grid=(NBLK,),
    in_specs=[
        pl.BlockSpec((BN, D), lambda i: (i, 0)),
        pl.BlockSpec((D, H), lambda i: (0, 0)),
        pl.BlockSpec((NC, BN, DW), lambda i: (0, i, 0)),
    ],
    out_specs=[
        pl.BlockSpec((BN, HH), lambda i: (i, 0)),
        pl.BlockSpec((BN, HH), lambda i: (i, 0)),
    ],
    out_shape=[
        jax.ShapeDtypeStruct((N, HH), jnp.bfloat16),
        jax.ShapeDtypeStruct((N, HH), jnp.bfloat16),
    ],
)


def _head_body(pa_ref, pb_ref, dp_ref, batch_ref, bgb_ref,
               wc_ref, bc_ref, out_ref, ssum, ssq, pooled):
    i = pl.program_id(0)
    deg = dp_ref[0, :, 0:1] + dp_ref[1, :, 0:1] + 1.0
    dinv = lax.rsqrt(deg)
    p = jnp.concatenate([pa_ref[...], pb_ref[...]],
                        axis=1).astype(jnp.float32)
    z = dinv * p + bgb_ref[0:1, :]

    @pl.when(i == 0)
    def _():
        ssum[...] = jnp.zeros_like(ssum)
        ssq[...] = jnp.zeros_like(ssq)
        pooled[...] = jnp.zeros_like(pooled)

    @pl.when(i < NBLK)
    def _():
        ssum[...] += jnp.sum(z, axis=0, keepdims=True)
        ssq[...] += jnp.sum(z * z, axis=0, keepdims=True)

    @pl.when(i >= NBLK)
    def _():
        mean = ssum[...] * (1.0 / N)
        var = ssq[...] * (1.0 / N) - mean * mean
        hn = ((z - mean) * lax.rsqrt(var + EPS) * bgb_ref[1:2, :]
              + bgb_ref[2:3, :])
        h2 = jnp.maximum(hn, 0.0)
        gid = lax.broadcasted_iota(jnp.int32, (G, 1), 0)
        oht = (gid == batch_ref[0]).astype(jnp.float32)
        pooled[...] += lax.dot_general(
            oht, h2, (((1,), (0,)), ((), ())),
            preferred_element_type=jnp.float32,
            precision=lax.Precision.HIGHEST)

    @pl.when(i == 2 * NBLK - 1)
    def _():
        out_ref[...] = jnp.dot(
            pooled[...], wc_ref[...], preferred_element_type=jnp.float32,
            precision=lax.Precision.HIGHEST) + bc_ref[...]


_head = pl.pallas_call(
    _head_body,
    grid=(2 * NBLK,),
    in_specs=[
        pl.BlockSpec((BN, HH), lambda i: (i % NBLK, 0)),
        pl.BlockSpec((BN, HH), lambda i: (i % NBLK, 0)),
        pl.BlockSpec((NC, BN, DW), lambda i: (0, i % NBLK, 0)),
        pl.BlockSpec((1, 1, BN), lambda i: (i % NBLK, 0, 0)),
        pl.BlockSpec((3, H), lambda i: (0, 0)),
        pl.BlockSpec((H, C), lambda i: (0, 0)),
        pl.BlockSpec((1, C), lambda i: (0, 0)),
    ],
    out_specs=pl.BlockSpec((G, C), lambda i: (0, 0)),
    out_shape=jax.ShapeDtypeStruct((G, C), jnp.float32),
    scratch_shapes=[
        pltpu.VMEM((1, H), jnp.float32),
        pltpu.VMEM((1, H), jnp.float32),
        pltpu.VMEM((G, H), jnp.float32),
    ],
)


def kernel(x, edge_index, batch, W1, b1, gamma, beta, Wc, bc):
    srcd = edge_index[0].reshape(NS, ACH, K)
    dstd = edge_index[1].reshape(NS, ACH, K)
    ones_d = jnp.ones((K, DW), jnp.float32)
    zeros_d = jnp.zeros((LAST, DW), jnp.float32)
    bgb = jnp.stack([b1, gamma, beta])

    dpart = _deg_kernel(dstd, ones_d, zeros_d)
    ga, gb = _matmul_scale(x, W1, dpart)
    pa, pb = _agg_kernel(ga, gb, srcd, dstd)
    logits = _head(pa, pb, dpart, batch.reshape(NBLK, 1, BN), bgb,
                   Wc, bc.reshape(1, C))
    return logits


# speedup vs baseline: 1.0042x; 1.0042x over previous
import functools

import jax
import jax.numpy as jnp
from jax import lax
from jax.experimental import pallas as pl
from jax.experimental.pallas import tpu as pltpu
from jax.experimental.pallas import tpu_sc as plsc

N = 10000
E = 320000
D = 128
H = 128
C = 16
G = 64
EPS = 1e-5

NC = 2
NS = 16
NW = NC * NS
EPW = E // NW
K = 100
NCH = EPW // K
HH = H // 2
EPT = E // NS
ACH = EPT // K
STR = 624
LAST = N - (NS - 1) * STR
DW = 16

_mesh = plsc.VectorSubcoreMesh(core_axis_name="c", subcore_axis_name="s")


@functools.partial(
    pl.kernel,
    out_type=jax.ShapeDtypeStruct((NC, N, DW), jnp.float32),
    mesh=_mesh,
    scratch_types=[
        pltpu.VMEM((NCH, K), jnp.int32),
        pltpu.VMEM((K, DW), jnp.float32),
        pltpu.VMEM_SHARED((N, DW), jnp.float32),
        pltpu.SemaphoreType.DMA,
        pltpu.SemaphoreType.DMA,
    ],
    compiler_params=pltpu.CompilerParams(use_tc_tiling_on_sc=False),
)
def _deg_kernel(dst_hbm, ones_hbm, zeros_hbm, dpart_hbm, dst_v, ones_v, accd,
                s0, s1):
    cid = lax.axis_index("c")
    sid = lax.axis_index("s")
    row0 = sid * STR
    pltpu.sync_copy(dst_hbm.at[sid, pl.ds(cid * NCH, NCH)], dst_v)
    pltpu.sync_copy(ones_hbm, ones_v)

    @pl.when(sid < NS - 1)
    def _():
        pltpu.sync_copy(zeros_hbm.at[pl.ds(0, STR)],
                        accd.at[pl.ds(row0, STR)])

    @pl.when(sid == NS - 1)
    def _():
        pltpu.sync_copy(zeros_hbm, accd.at[pl.ds((NS - 1) * STR, LAST)])

    plsc.subcore_barrier()

    def body(jj, carry):
        c0 = pltpu.async_copy(ones_v, accd.at[dst_v.at[2 * jj]], s0, add=True)
        c1 = pltpu.async_copy(ones_v, accd.at[dst_v.at[2 * jj + 1]], s1,
                              add=True)
        c0.wait()
        c1.wait()
        return carry

    lax.fori_loop(0, NCH // 2, body, 0)
    plsc.subcore_barrier()

    @pl.when(sid < NS - 1)
    def _():
        pltpu.sync_copy(accd.at[pl.ds(row0, STR)],
                        dpart_hbm.at[cid, pl.ds(row0, STR)])

    @pl.when(sid == NS - 1)
    def _():
        pltpu.sync_copy(accd.at[pl.ds((NS - 1) * STR, LAST)],
                        dpart_hbm.at[cid, pl.ds((NS - 1) * STR, LAST)])


@functools.partial(
    pl.kernel,
    out_type=(jax.ShapeDtypeStruct((N, HH), jnp.bfloat16),
              jax.ShapeDtypeStruct((N, HH), jnp.bfloat16)),
    mesh=_mesh,
    scratch_types=[
        pltpu.VMEM((ACH, K), jnp.int32),
        pltpu.VMEM((ACH, K), jnp.int32),
    ] + [pltpu.VMEM((K, HH), jnp.bfloat16)] * 5
      + [pltpu.VMEM_SHARED((N, HH), jnp.bfloat16)]
      + [pltpu.SemaphoreType.DMA] * 10,
    compiler_params=pltpu.CompilerParams(use_tc_tiling_on_sc=False),
)
def _agg_kernel(ga_hbm, gb_hbm, src_hbm, dst_hbm,
                pa_hbm, pb_hbm,
                src_v, dst_v, *rest):
    bufs = rest[0:5]
    acc = rest[5]
    gsems = rest[6:11]
    ssems = rest[11:16]
    cid = lax.axis_index("c")
    sid = lax.axis_index("s")
    row0 = sid * STR
    pltpu.sync_copy(src_hbm.at[sid], src_v)
    pltpu.sync_copy(dst_hbm.at[sid], dst_v)

    @pl.when((cid == 0) & (sid < NS - 1))
    def _():
        pltpu.sync_copy(ga_hbm.at[pl.ds(row0, STR)], acc.at[pl.ds(row0, STR)])

    @pl.when((cid == 0) & (sid == NS - 1))
    def _():
        pltpu.sync_copy(ga_hbm.at[pl.ds((NS - 1) * STR, LAST)],
                        acc.at[pl.ds((NS - 1) * STR, LAST)])

    @pl.when((cid != 0) & (sid < NS - 1))
    def _():
        pltpu.sync_copy(gb_hbm.at[pl.ds(row0, STR)], acc.at[pl.ds(row0, STR)])

    @pl.when((cid != 0) & (sid == NS - 1))
    def _():
        pltpu.sync_copy(gb_hbm.at[pl.ds((NS - 1) * STR, LAST)],
                        acc.at[pl.ds((NS - 1) * STR, LAST)])

    plsc.subcore_barrier()

    NB = 5

    def gather(j, buf, sem):
        @pl.when(cid == 0)
        def _():
            pltpu.async_copy(ga_hbm.at[src_v.at[j]], buf, sem)

        @pl.when(cid != 0)
        def _():
            pltpu.async_copy(gb_hbm.at[src_v.at[j]], buf, sem)

    for b in range(NB):
        gather(b, bufs[b], gsems[b])

    def body(jj, carry):
        scs = []
        for b in range(NB):
            j = NB * jj + b
            pltpu.make_async_copy(ga_hbm.at[src_v.at[0]], bufs[b],
                                  gsems[b]).wait()
            scs.append(pltpu.async_copy(bufs[b], acc.at[dst_v.at[j]],
                                        ssems[b], add=True))
        for b in range(NB):
            j = NB * jj + b
            scs[b].wait()

            @pl.when(j + NB < ACH)
            def _(b=b, j=j):
                gather(j + NB, bufs[b], gsems[b])

        return carry

    lax.fori_loop(0, ACH // NB, body, 0)
    plsc.subcore_barrier()

    def writeback(out_hbm):
        @pl.when(sid < NS - 1)
        def _():
            pltpu.sync_copy(acc.at[pl.ds(row0, STR)],
                            out_hbm.at[pl.ds(row0, STR)])

        @pl.when(sid == NS - 1)
        def _():
            pltpu.sync_copy(acc.at[pl.ds((NS - 1) * STR, LAST)],
                            out_hbm.at[pl.ds((NS - 1) * STR, LAST)])

    @pl.when(cid == 0)
    def _():
        writeback(pa_hbm)

    @pl.when(cid != 0)
    def _():
        writeback(pb_hbm)


BN = 2000
NBLK = N // BN


def _mm_body(x_ref, w_ref, dp_ref, ga_ref, gb_ref):
    deg = dp_ref[0, :, 0:1] + dp_ref[1, :, 0:1] + 1.0
    dinv = lax.rsqrt(deg)
    h = jnp.dot(x_ref[...], w_ref[...], preferred_element_type=jnp.float32,
                precision=lax.Precision.HIGHEST)
    g = (h * dinv).astype(jnp.bfloat16)
    ga_ref[...] = g[:, :HH]
    gb_ref[...] = g[:, HH:]


_matmul_scale = pl.pallas_call(
    _mm_body,
    grid=(NBLK,),
    in_specs=[
        pl.BlockSpec((BN, D), lambda i: (i, 0)),
        pl.BlockSpec((D, H), lambda i: (0, 0)),
        pl.BlockSpec((NC, BN, DW), lambda i: (0, i, 0)),
    ],
    out_specs=[
        pl.BlockSpec((BN, HH), lambda i: (i, 0)),
        pl.BlockSpec((BN, HH), lambda i: (i, 0)),
    ],
    out_shape=[
        jax.ShapeDtypeStruct((N, HH), jnp.bfloat16),
        jax.ShapeDtypeStruct((N, HH), jnp.bfloat16),
    ],
)


def _head_body(pa_ref, pb_ref, dp_ref, batch_ref, bgb_ref,
               wc_ref, bc_ref, out_ref, ssum, ssq, pooled):
    i = pl.program_id(0)
    deg = dp_ref[0, :, 0:1] + dp_ref[1, :, 0:1] + 1.0
    dinv = lax.rsqrt(deg)
    p = jnp.concatenate([pa_ref[...], pb_ref[...]],
                        axis=1).astype(jnp.float32)
    z = dinv * p + bgb_ref[0:1, :]

    @pl.when(i == 0)
    def _():
        ssum[...] = jnp.zeros_like(ssum)
        ssq[...] = jnp.zeros_like(ssq)
        pooled[...] = jnp.zeros_like(pooled)

    @pl.when(i < NBLK)
    def _():
        ssum[...] += jnp.sum(z, axis=0, keepdims=True)
        ssq[...] += jnp.sum(z * z, axis=0, keepdims=True)

    @pl.when(i >= NBLK)
    def _():
        mean = ssum[...] * (1.0 / N)
        var = ssq[...] * (1.0 / N) - mean * mean
        hn = ((z - mean) * lax.rsqrt(var + EPS) * bgb_ref[1:2, :]
              + bgb_ref[2:3, :])
        h2 = jnp.maximum(hn, 0.0)
        gid = lax.broadcasted_iota(jnp.int32, (G, 1), 0)
        oht = (gid == batch_ref[0]).astype(jnp.float32)
        pooled[...] += lax.dot_general(
            oht, h2, (((1,), (0,)), ((), ())),
            preferred_element_type=jnp.float32,
            precision=lax.Precision.HIGHEST)

    @pl.when(i == 2 * NBLK - 1)
    def _():
        out_ref[...] = jnp.dot(
            pooled[...], wc_ref[...], preferred_element_type=jnp.float32,
            precision=lax.Precision.HIGHEST) + bc_ref[...]


_head = pl.pallas_call(
    _head_body,
    grid=(2 * NBLK,),
    in_specs=[
        pl.BlockSpec((BN, HH), lambda i: (i % NBLK, 0)),
        pl.BlockSpec((BN, HH), lambda i: (i % NBLK, 0)),
        pl.BlockSpec((NC, BN, DW), lambda i: (0, i % NBLK, 0)),
        pl.BlockSpec((1, 1, BN), lambda i: (i % NBLK, 0, 0)),
        pl.BlockSpec((3, H), lambda i: (0, 0)),
        pl.BlockSpec((H, C), lambda i: (0, 0)),
        pl.BlockSpec((1, C), lambda i: (0, 0)),
    ],
    out_specs=pl.BlockSpec((G, C), lambda i: (0, 0)),
    out_shape=jax.ShapeDtypeStruct((G, C), jnp.float32),
    scratch_shapes=[
        pltpu.VMEM((1, H), jnp.float32),
        pltpu.VMEM((1, H), jnp.float32),
        pltpu.VMEM((G, H), jnp.float32),
    ],
)


def kernel(x, edge_index, batch, W1, b1, gamma, beta, Wc, bc):
    srcd = edge_index[0].reshape(NS, ACH, K)
    dstd = edge_index[1].reshape(NS, ACH, K)
    ones_d = jnp.ones((K, DW), jnp.float32)
    zeros_d = jnp.zeros((LAST, DW), jnp.float32)
    bgb = jnp.stack([b1, gamma, beta])

    dpart = _deg_kernel(dstd, ones_d, zeros_d)
    ga, gb = _matmul_scale(x, W1, dpart)
    pa, pb = _agg_kernel(ga, gb, srcd, dstd)
    logits = _head(pa, pb, dpart, batch.reshape(NBLK, 1, BN), bgb,
                   Wc, bc.reshape(1, C))
    return logits


# speedup vs baseline: 1.0223x; 1.0181x over previous
import functools

import jax
import jax.numpy as jnp
from jax import lax
from jax.experimental import pallas as pl
from jax.experimental.pallas import tpu as pltpu
from jax.experimental.pallas import tpu_sc as plsc

N = 10000
E = 320000
D = 128
H = 128
C = 16
G = 64
EPS = 1e-5

NC = 2
NS = 16
NW = NC * NS
EPW = E // NW
K = 100
NCH = EPW // K
HH = H // 2
EPT = E // NS
ACH = EPT // K
STR = 624
LAST = N - (NS - 1) * STR
DW = 16

_mesh = plsc.VectorSubcoreMesh(core_axis_name="c", subcore_axis_name="s")


@functools.partial(
    pl.kernel,
    out_type=jax.ShapeDtypeStruct((NC, N, DW), jnp.float32),
    mesh=_mesh,
    scratch_types=[
        pltpu.VMEM((NCH, K), jnp.int32),
        pltpu.VMEM((K, DW), jnp.float32),
        pltpu.VMEM_SHARED((N, DW), jnp.float32),
        pltpu.SemaphoreType.DMA,
        pltpu.SemaphoreType.DMA,
    ],
    compiler_params=pltpu.CompilerParams(use_tc_tiling_on_sc=False),
)
def _deg_kernel(e4_hbm, ones_hbm, zeros_hbm, dpart_hbm, dst_v, ones_v, accd,
                s0, s1):
    cid = lax.axis_index("c")
    sid = lax.axis_index("s")
    row0 = sid * STR
    pltpu.sync_copy(e4_hbm.at[1, sid, pl.ds(cid * NCH, NCH)], dst_v)
    pltpu.sync_copy(ones_hbm, ones_v)

    @pl.when(sid < NS - 1)
    def _():
        pltpu.sync_copy(zeros_hbm.at[pl.ds(0, STR)],
                        accd.at[pl.ds(row0, STR)])

    @pl.when(sid == NS - 1)
    def _():
        pltpu.sync_copy(zeros_hbm, accd.at[pl.ds((NS - 1) * STR, LAST)])

    plsc.subcore_barrier()

    def body(jj, carry):
        c0 = pltpu.async_copy(ones_v, accd.at[dst_v.at[2 * jj]], s0, add=True)
        c1 = pltpu.async_copy(ones_v, accd.at[dst_v.at[2 * jj + 1]], s1,
                              add=True)
        c0.wait()
        c1.wait()
        return carry

    lax.fori_loop(0, NCH // 2, body, 0)
    plsc.subcore_barrier()

    @pl.when(sid < NS - 1)
    def _():
        pltpu.sync_copy(accd.at[pl.ds(row0, STR)],
                        dpart_hbm.at[cid, pl.ds(row0, STR)])

    @pl.when(sid == NS - 1)
    def _():
        pltpu.sync_copy(accd.at[pl.ds((NS - 1) * STR, LAST)],
                        dpart_hbm.at[cid, pl.ds((NS - 1) * STR, LAST)])


@functools.partial(
    pl.kernel,
    out_type=(jax.ShapeDtypeStruct((N, HH), jnp.bfloat16),
              jax.ShapeDtypeStruct((N, HH), jnp.bfloat16)),
    mesh=_mesh,
    scratch_types=[
        pltpu.VMEM((ACH, K), jnp.int32),
        pltpu.VMEM((ACH, K), jnp.int32),
    ] + [pltpu.VMEM((K, HH), jnp.bfloat16)] * 5
      + [pltpu.VMEM_SHARED((N, HH), jnp.bfloat16)]
      + [pltpu.SemaphoreType.DMA] * 10,
    compiler_params=pltpu.CompilerParams(use_tc_tiling_on_sc=False),
)
def _agg_kernel(ga_hbm, gb_hbm, e4_hbm,
                pa_hbm, pb_hbm,
                src_v, dst_v, *rest):
    bufs = rest[0:5]
    acc = rest[5]
    gsems = rest[6:11]
    ssems = rest[11:16]
    cid = lax.axis_index("c")
    sid = lax.axis_index("s")
    row0 = sid * STR
    pltpu.sync_copy(e4_hbm.at[0, sid], src_v)
    pltpu.sync_copy(e4_hbm.at[1, sid], dst_v)

    @pl.when((cid == 0) & (sid < NS - 1))
    def _():
        pltpu.sync_copy(ga_hbm.at[pl.ds(row0, STR)], acc.at[pl.ds(row0, STR)])

    @pl.when((cid == 0) & (sid == NS - 1))
    def _():
        pltpu.sync_copy(ga_hbm.at[pl.ds((NS - 1) * STR, LAST)],
                        acc.at[pl.ds((NS - 1) * STR, LAST)])

    @pl.when((cid != 0) & (sid < NS - 1))
    def _():
        pltpu.sync_copy(gb_hbm.at[pl.ds(row0, STR)], acc.at[pl.ds(row0, STR)])

    @pl.when((cid != 0) & (sid == NS - 1))
    def _():
        pltpu.sync_copy(gb_hbm.at[pl.ds((NS - 1) * STR, LAST)],
                        acc.at[pl.ds((NS - 1) * STR, LAST)])

    plsc.subcore_barrier()

    NB = 5

    def gather(j, buf, sem):
        @pl.when(cid == 0)
        def _():
            pltpu.async_copy(ga_hbm.at[src_v.at[j]], buf, sem)

        @pl.when(cid != 0)
        def _():
            pltpu.async_copy(gb_hbm.at[src_v.at[j]], buf, sem)

    for b in range(NB):
        gather(b, bufs[b], gsems[b])

    def body(jj, carry):
        scs = []
        for b in range(NB):
            j = NB * jj + b
            pltpu.make_async_copy(ga_hbm.at[src_v.at[0]], bufs[b],
                                  gsems[b]).wait()
            scs.append(pltpu.async_copy(bufs[b], acc.at[dst_v.at[j]],
                                        ssems[b], add=True))
        for b in range(NB):
            j = NB * jj + b
            scs[b].wait()

            @pl.when(j + NB < ACH)
            def _(b=b, j=j):
                gather(j + NB, bufs[b], gsems[b])

        return carry

    lax.fori_loop(0, ACH // NB, body, 0)
    plsc.subcore_barrier()

    def writeback(out_hbm):
        @pl.when(sid < NS - 1)
        def _():
            pltpu.sync_copy(acc.at[pl.ds(row0, STR)],
                            out_hbm.at[pl.ds(row0, STR)])

        @pl.when(sid == NS - 1)
        def _():
            pltpu.sync_copy(acc.at[pl.ds((NS - 1) * STR, LAST)],
                            out_hbm.at[pl.ds((NS - 1) * STR, LAST)])

    @pl.when(cid == 0)
    def _():
        writeback(pa_hbm)

    @pl.when(cid != 0)
    def _():
        writeback(pb_hbm)


BN = 2000
NBLK = N // BN


def _mm_body(x_ref, w_ref, dp_ref, ga_ref, gb_ref):
    deg = dp_ref[0, :, 0:1] + dp_ref[1, :, 0:1] + 1.0
    dinv = lax.rsqrt(deg)
    h = jnp.dot(x_ref[...], w_ref[...], preferred_element_type=jnp.float32,
                precision=lax.Precision.HIGHEST)
    g = (h * dinv).astype(jnp.bfloat16)
    ga_ref[...] = g[:, :HH]
    gb_ref[...] = g[:, HH:]


_matmul_scale = pl.pallas_call(
    _mm_body,
    grid=(NBLK,),
    in_specs=[
        pl.BlockSpec((BN, D), lambda i: (i, 0)),
        pl.BlockSpec((D, H), lambda i: (0, 0)),
        pl.BlockSpec((NC, BN, DW), lambda i: (0, i, 0)),
    ],
    out_specs=[
        pl.BlockSpec((BN, HH), lambda i: (i, 0)),
        pl.BlockSpec((BN, HH), lambda i: (i, 0)),
    ],
    out_shape=[
        jax.ShapeDtypeStruct((N, HH), jnp.bfloat16),
        jax.ShapeDtypeStruct((N, HH), jnp.bfloat16),
    ],
)


def _head_body(pa_ref, pb_ref, dp_ref, batch_ref, bgb_ref,
               wc_ref, bc_ref, out_ref, ssum, ssq, pooled):
    i = pl.program_id(0)
    deg = dp_ref[0, :, 0:1] + dp_ref[1, :, 0:1] + 1.0
    dinv = lax.rsqrt(deg)
    p = jnp.concatenate([pa_ref[...], pb_ref[...]],
                        axis=1).astype(jnp.float32)
    z = dinv * p + bgb_ref[0:1, :]

    @pl.when(i == 0)
    def _():
        ssum[...] = jnp.zeros_like(ssum)
        ssq[...] = jnp.zeros_like(ssq)
        pooled[...] = jnp.zeros_like(pooled)

    @pl.when(i < NBLK)
    def _():
        ssum[...] += jnp.sum(z, axis=0, keepdims=True)
        ssq[...] += jnp.sum(z * z, axis=0, keepdims=True)

    @pl.when(i >= NBLK)
    def _():
        mean = ssum[...] * (1.0 / N)
        var = ssq[...] * (1.0 / N) - mean * mean
        hn = ((z - mean) * lax.rsqrt(var + EPS) * bgb_ref[1:2, :]
              + bgb_ref[2:3, :])
        h2 = jnp.maximum(hn, 0.0)
        gid = lax.broadcasted_iota(jnp.int32, (G, 1), 0)
        oht = (gid == batch_ref[0]).astype(jnp.float32)
        pooled[...] += lax.dot_general(
            oht, h2, (((1,), (0,)), ((), ())),
            preferred_element_type=jnp.float32,
            precision=lax.Precision.HIGHEST)

    @pl.when(i == 2 * NBLK - 1)
    def _():
        out_ref[...] = jnp.dot(
            pooled[...], wc_ref[...], preferred_element_type=jnp.float32,
            precision=lax.Precision.HIGHEST) + bc_ref[...]


_head = pl.pallas_call(
    _head_body,
    grid=(2 * NBLK,),
    in_specs=[
        pl.BlockSpec((BN, HH), lambda i: (i % NBLK, 0)),
        pl.BlockSpec((BN, HH), lambda i: (i % NBLK, 0)),
        pl.BlockSpec((NC, BN, DW), lambda i: (0, i % NBLK, 0)),
        pl.BlockSpec((1, 1, BN), lambda i: (i % NBLK, 0, 0)),
        pl.BlockSpec((3, H), lambda i: (0, 0)),
        pl.BlockSpec((H, C), lambda i: (0, 0)),
        pl.BlockSpec((1, C), lambda i: (0, 0)),
    ],
    out_specs=pl.BlockSpec((G, C), lambda i: (0, 0)),
    out_shape=jax.ShapeDtypeStruct((G, C), jnp.float32),
    scratch_shapes=[
        pltpu.VMEM((1, H), jnp.float32),
        pltpu.VMEM((1, H), jnp.float32),
        pltpu.VMEM((G, H), jnp.float32),
    ],
)


def kernel(x, edge_index, batch, W1, b1, gamma, beta, Wc, bc):
    e4 = edge_index.reshape(2, NS, ACH, K)
    ones_d = jnp.ones((K, DW), jnp.float32)
    zeros_d = jnp.zeros((LAST, DW), jnp.float32)
    bgb = jnp.stack([b1, gamma, beta])

    dpart = _deg_kernel(e4, ones_d, zeros_d)
    ga, gb = _matmul_scale(x, W1, dpart)
    pa, pb = _agg_kernel(ga, gb, e4)
    logits = _head(pa, pb, dpart, batch.reshape(NBLK, 1, BN), bgb,
                   Wc, bc.reshape(1, C))
    return logits


# speedup vs baseline: 1.0464x; 1.0236x over previous
import functools

import jax
import jax.numpy as jnp
from jax import lax
from jax.experimental import pallas as pl
from jax.experimental.pallas import tpu as pltpu
from jax.experimental.pallas import tpu_sc as plsc

N = 10000
E = 320000
D = 128
H = 128
C = 16
G = 64
EPS = 1e-5

NC = 2
NS = 16
NW = NC * NS
EPW = E // NW
K = 100
NCH = EPW // K
HH = H // 2
EPT = E // NS
ACH = EPT // K
STR = 624
LAST = N - (NS - 1) * STR
DW = 16

_mesh = plsc.VectorSubcoreMesh(core_axis_name="c", subcore_axis_name="s")


@functools.partial(
    pl.kernel,
    out_type=jax.ShapeDtypeStruct((NC, N, DW), jnp.float32),
    mesh=_mesh,
    scratch_types=[
        pltpu.VMEM((NCH, K), jnp.int32),
        pltpu.VMEM((K, DW), jnp.float32),
        pltpu.VMEM_SHARED((N, DW), jnp.float32),
        pltpu.SemaphoreType.DMA,
        pltpu.SemaphoreType.DMA,
    ],
    compiler_params=pltpu.CompilerParams(use_tc_tiling_on_sc=False),
)
def _deg_kernel(e4_hbm, ones_hbm, zeros_hbm, dpart_hbm, dst_v, ones_v, accd,
                s0, s1):
    cid = lax.axis_index("c")
    sid = lax.axis_index("s")
    row0 = sid * STR
    pltpu.sync_copy(e4_hbm.at[1, sid, pl.ds(cid * NCH, NCH)], dst_v)
    pltpu.sync_copy(ones_hbm, ones_v)

    @pl.when(sid < NS - 1)
    def _():
        pltpu.sync_copy(zeros_hbm.at[pl.ds(0, STR)],
                        accd.at[pl.ds(row0, STR)])

    @pl.when(sid == NS - 1)
    def _():
        pltpu.sync_copy(zeros_hbm, accd.at[pl.ds((NS - 1) * STR, LAST)])

    plsc.subcore_barrier()

    def body(jj, carry):
        c0 = pltpu.async_copy(ones_v, accd.at[dst_v.at[2 * jj]], s0, add=True)
        c1 = pltpu.async_copy(ones_v, accd.at[dst_v.at[2 * jj + 1]], s1,
                              add=True)
        c0.wait()
        c1.wait()
        return carry

    lax.fori_loop(0, NCH // 2, body, 0)
    plsc.subcore_barrier()

    @pl.when(sid < NS - 1)
    def _():
        pltpu.sync_copy(accd.at[pl.ds(row0, STR)],
                        dpart_hbm.at[cid, pl.ds(row0, STR)])

    @pl.when(sid == NS - 1)
    def _():
        pltpu.sync_copy(accd.at[pl.ds((NS - 1) * STR, LAST)],
                        dpart_hbm.at[cid, pl.ds((NS - 1) * STR, LAST)])


@functools.partial(
    pl.kernel,
    out_type=(jax.ShapeDtypeStruct((N, HH), jnp.bfloat16),
              jax.ShapeDtypeStruct((N, HH), jnp.bfloat16)),
    mesh=_mesh,
    scratch_types=[
        pltpu.VMEM((ACH, K), jnp.int32),
        pltpu.VMEM((ACH, K), jnp.int32),
    ] + [pltpu.VMEM((K, HH), jnp.bfloat16)] * 10
      + [pltpu.VMEM_SHARED((N, HH), jnp.bfloat16)]
      + [pltpu.SemaphoreType.DMA] * 20,
    compiler_params=pltpu.CompilerParams(use_tc_tiling_on_sc=False),
)
def _agg_kernel(ga_hbm, gb_hbm, e4_hbm,
                pa_hbm, pb_hbm,
                src_v, dst_v, *rest):
    bufs = rest[0:10]
    acc = rest[10]
    gsems = rest[11:21]
    ssems = rest[21:31]
    cid = lax.axis_index("c")
    sid = lax.axis_index("s")
    row0 = sid * STR
    pltpu.sync_copy(e4_hbm.at[0, sid], src_v)
    pltpu.sync_copy(e4_hbm.at[1, sid], dst_v)

    @pl.when((cid == 0) & (sid < NS - 1))
    def _():
        pltpu.sync_copy(ga_hbm.at[pl.ds(row0, STR)], acc.at[pl.ds(row0, STR)])

    @pl.when((cid == 0) & (sid == NS - 1))
    def _():
        pltpu.sync_copy(ga_hbm.at[pl.ds((NS - 1) * STR, LAST)],
                        acc.at[pl.ds((NS - 1) * STR, LAST)])

    @pl.when((cid != 0) & (sid < NS - 1))
    def _():
        pltpu.sync_copy(gb_hbm.at[pl.ds(row0, STR)], acc.at[pl.ds(row0, STR)])

    @pl.when((cid != 0) & (sid == NS - 1))
    def _():
        pltpu.sync_copy(gb_hbm.at[pl.ds((NS - 1) * STR, LAST)],
                        acc.at[pl.ds((NS - 1) * STR, LAST)])

    plsc.subcore_barrier()

    NB = 10

    def gather(j, buf, sem):
        @pl.when(cid == 0)
        def _():
            pltpu.async_copy(ga_hbm.at[src_v.at[j]], buf, sem)

        @pl.when(cid != 0)
        def _():
            pltpu.async_copy(gb_hbm.at[src_v.at[j]], buf, sem)

    for b in range(NB):
        gather(b, bufs[b], gsems[b])

    def body(jj, carry):
        scs = []
        for b in range(NB):
            j = NB * jj + b
            pltpu.make_async_copy(ga_hbm.at[src_v.at[0]], bufs[b],
                                  gsems[b]).wait()
            scs.append(pltpu.async_copy(bufs[b], acc.at[dst_v.at[j]],
                                        ssems[b], add=True))
        for b in range(NB):
            j = NB * jj + b
            scs[b].wait()

            @pl.when(j + NB < ACH)
            def _(b=b, j=j):
                gather(j + NB, bufs[b], gsems[b])

        return carry

    lax.fori_loop(0, ACH // NB, body, 0)
    plsc.subcore_barrier()

    def writeback(out_hbm):
        @pl.when(sid < NS - 1)
        def _():
            pltpu.sync_copy(acc.at[pl.ds(row0, STR)],
                            out_hbm.at[pl.ds(row0, STR)])

        @pl.when(sid == NS - 1)
        def _():
            pltpu.sync_copy(acc.at[pl.ds((NS - 1) * STR, LAST)],
                            out_hbm.at[pl.ds((NS - 1) * STR, LAST)])

    @pl.when(cid == 0)
    def _():
        writeback(pa_hbm)

    @pl.when(cid != 0)
    def _():
        writeback(pb_hbm)


BN = 2000
NBLK = N // BN


def _mm_body(x_ref, w_ref, dp_ref, ga_ref, gb_ref):
    deg = dp_ref[0, :, 0:1] + dp_ref[1, :, 0:1] + 1.0
    dinv = lax.rsqrt(deg)
    h = jnp.dot(x_ref[...], w_ref[...], preferred_element_type=jnp.float32,
                precision=lax.Precision.HIGHEST)
    g = (h * dinv).astype(jnp.bfloat16)
    ga_ref[...] = g[:, :HH]
    gb_ref[...] = g[:, HH:]


_matmul_scale = pl.pallas_call(
    _mm_body,
    grid=(NBLK,),
    in_specs=[
        pl.BlockSpec((BN, D), lambda i: (i, 0)),
        pl.BlockSpec((D, H), lambda i: (0, 0)),
        pl.BlockSpec((NC, BN, DW), lambda i: (0, i, 0)),
    ],
    out_specs=[
        pl.BlockSpec((BN, HH), lambda i: (i, 0)),
        pl.BlockSpec((BN, HH), lambda i: (i, 0)),
    ],
    out_shape=[
        jax.ShapeDtypeStruct((N, HH), jnp.bfloat16),
        jax.ShapeDtypeStruct((N, HH), jnp.bfloat16),
    ],
)


def _head_body(pa_ref, pb_ref, dp_ref, batch_ref, bgb_ref,
               wc_ref, bc_ref, out_ref, ssum, ssq, pooled):
    i = pl.program_id(0)
    deg = dp_ref[0, :, 0:1] + dp_ref[1, :, 0:1] + 1.0
    dinv = lax.rsqrt(deg)
    p = jnp.concatenate([pa_ref[...], pb_ref[...]],
                        axis=1).astype(jnp.float32)
    z = dinv * p + bgb_ref[0:1, :]

    @pl.when(i == 0)
    def _():
        ssum[...] = jnp.zeros_like(ssum)
        ssq[...] = jnp.zeros_like(ssq)
        pooled[...] = jnp.zeros_like(pooled)

    @pl.when(i < NBLK)
    def _():
        ssum[...] += jnp.sum(z, axis=0, keepdims=True)
        ssq[...] += jnp.sum(z * z, axis=0, keepdims=True)

    @pl.when(i >= NBLK)
    def _():
        mean = ssum[...] * (1.0 / N)
        var = ssq[...] * (1.0 / N) - mean * mean
        hn = ((z - mean) * lax.rsqrt(var + EPS) * bgb_ref[1:2, :]
              + bgb_ref[2:3, :])
        h2 = jnp.maximum(hn, 0.0)
        gid = lax.broadcasted_iota(jnp.int32, (G, 1), 0)
        oht = (gid == batch_ref[0]).astype(jnp.float32)
        pooled[...] += lax.dot_general(
            oht, h2, (((1,), (0,)), ((), ())),
            preferred_element_type=jnp.float32,
            precision=lax.Precision.HIGHEST)

    @pl.when(i == 2 * NBLK - 1)
    def _():
        out_ref[...] = jnp.dot(
            pooled[...], wc_ref[...], preferred_element_type=jnp.float32,
            precision=lax.Precision.HIGHEST) + bc_ref[...]


_head = pl.pallas_call(
    _head_body,
    grid=(2 * NBLK,),
    in_specs=[
        pl.BlockSpec((BN, HH), lambda i: (i % NBLK, 0)),
        pl.BlockSpec((BN, HH), lambda i: (i % NBLK, 0)),
        pl.BlockSpec((NC, BN, DW), lambda i: (0, i % NBLK, 0)),
        pl.BlockSpec((1, 1, BN), lambda i: (i % NBLK, 0, 0)),
        pl.BlockSpec((3, H), lambda i: (0, 0)),
        pl.BlockSpec((H, C), lambda i: (0, 0)),
        pl.BlockSpec((1, C), lambda i: (0, 0)),
    ],
    out_specs=pl.BlockSpec((G, C), lambda i: (0, 0)),
    out_shape=jax.ShapeDtypeStruct((G, C), jnp.float32),
    scratch_shapes=[
        pltpu.VMEM((1, H), jnp.float32),
        pltpu.VMEM((1, H), jnp.float32),
        pltpu.VMEM((G, H), jnp.float32),
    ],
)


def kernel(x, edge_index, batch, W1, b1, gamma, beta, Wc, bc):
    e4 = edge_index.reshape(2, NS, ACH, K)
    ones_d = jnp.ones((K, DW), jnp.float32)
    zeros_d = jnp.zeros((LAST, DW), jnp.float32)
    bgb = jnp.stack([b1, gamma, beta])

    dpart = _deg_kernel(e4, ones_d, zeros_d)
    ga, gb = _matmul_scale(x, W1, dpart)
    pa, pb = _agg_kernel(ga, gb, e4)
    logits = _head(pa, pb, dpart, batch.reshape(NBLK, 1, BN), bgb,
                   Wc, bc.reshape(1, C))
    return logits
